# baseline (speedup 1.0000x reference)
import functools

import jax
import jax.numpy as jnp
from jax import lax
from jax.experimental import pallas as pl
from jax.experimental.pallas import tpu as pltpu
from jax.experimental.pallas import tpu_sc as plsc

K = 8192
E = 32
T = 4096
BT = 512
NT = T // BT
CC = 0.25


def _vq_body(z_ref, e_ref, idx_ref):
    e = e_ref[...]
    e2 = jnp.sum(e * e, axis=1)

    def chunk(i, carry):
        z = z_ref[pl.ds(i * BT, BT), :]
        mm = lax.dot_general(z * -2.0, e, (((1,), (1,)), ((), ())),
                             preferred_element_type=jnp.float32)
        z2 = jnp.sum(z * z, axis=1, keepdims=True)
        scores = z2 + e2[None, :] + mm
        idx = jnp.argmin(scores, axis=1).astype(jnp.int32)
        idx_ref[pl.ds(i * BT, BT)] = idx
        return carry

    lax.fori_loop(0, NT, chunk, 0)


_vq_call = pl.pallas_call(
    _vq_body,
    in_specs=[
        pl.BlockSpec((T, E), lambda: (0, 0)),
        pl.BlockSpec((K, E), lambda: (0, 0)),
    ],
    out_specs=[pl.BlockSpec((T,), lambda: (0,))],
    out_shape=[jax.ShapeDtypeStruct((T,), jnp.int32)],
    compiler_params=pltpu.CompilerParams(vmem_limit_bytes=117 * 1024 * 1024),
)


_NC, _NS = 2, 16
_TPS = T // _NS
KPT = K // _NS

_LOG2_POLY = (0.04342868488885992, -0.4048646480119264, 1.5938912717255367,
              -3.4924755918040757, 5.04685963461102, -2.7868074212476523)
_LN2 = 0.6931471805599453


@functools.cache
def _sc_call():
    mesh = plsc.VectorSubcoreMesh(
        core_axis_name="c", subcore_axis_name="s", num_cores=_NC)

    @functools.partial(
        pl.kernel,
        mesh=mesh,
        compiler_params=pltpu.CompilerParams(
            use_tc_tiling_on_sc=False, needs_layout_passes=False),
        out_type=(
            jax.ShapeDtypeStruct((T, E), jnp.float32),
            jax.ShapeDtypeStruct((16,), jnp.float32),
            jax.ShapeDtypeStruct((16,), jnp.float32),
        ),
        scratch_types=[
            pltpu.VMEM((_TPS,), jnp.int32),
            pltpu.VMEM((_TPS, E), jnp.float32),
            pltpu.VMEM((_TPS, E), jnp.float32),
            pltpu.VMEM((T,), jnp.int32),
            pltpu.VMEM((KPT,), jnp.float32),
            pltpu.VMEM((16,), jnp.float32),
            pltpu.VMEM((16, 16), jnp.float32),
            pltpu.VMEM_SHARED((16, 16), jnp.float32),
            pltpu.SemaphoreType.DMA,
        ],
    )
    def _sc_body(emb_hbm, idx_hbm, z_hbm, zq_hbm, loss_hbm, ppl_hbm,
                 idx_v, rows_v, zin_v, idxall_v, cnt_v, stage_v, part_v,
                 shared, sem):
        c = lax.axis_index("c")
        s = lax.axis_index("s")

        @pl.when(c == 1)
        def _():
            base = s * _TPS
            pltpu.sync_copy(idx_hbm.at[pl.ds(base, _TPS)], idx_v)
            pltpu.async_copy(emb_hbm.at[idx_v], rows_v, sem).wait()
            pltpu.sync_copy(rows_v, zq_hbm.at[pl.ds(base, _TPS)])
            pltpu.sync_copy(z_hbm.at[pl.ds(base, _TPS)], zin_v)

            def lbody(j, acc):
                d0 = rows_v[j, pl.ds(0, 16)] - zin_v[j, pl.ds(0, 16)]
                d1 = rows_v[j, pl.ds(16, 16)] - zin_v[j, pl.ds(16, 16)]
                return acc + d0 * d0 + d1 * d1

            acc = lax.fori_loop(0, _TPS, lbody, jnp.zeros((16,), jnp.float32))
            stage_v[...] = acc
            pltpu.sync_copy(stage_v, shared.at[s])
            plsc.subcore_barrier()

            @pl.when(s == 0)
            def _():
                pltpu.sync_copy(shared, part_v)
                tot = jnp.zeros((16,), jnp.float32)
                for r in range(16):
                    tot = tot + part_v[r, :]
                ssum = jnp.sum(tot)
                lv = jnp.zeros((16,), jnp.float32) + ssum * (CC / (T * E))
                stage_v[...] = lv
                pltpu.sync_copy(stage_v, loss_hbm)

        @pl.when(c == 0)
        def _():
            pltpu.sync_copy(idx_hbm, idxall_v)
            lo = s * KPT
            zeros = jnp.zeros((16,), jnp.float32)
            ones = jnp.full((16,), 1.0, jnp.float32)

            def zbody(j, carry):
                cnt_v[pl.ds(j * 16, 16)] = zeros
                return carry

            lax.fori_loop(0, KPT // 16, zbody, 0)

            def hbody(j, carry):
                iv = idxall_v[pl.ds(j * 16, 16)]
                m = (iv >= lo) & (iv < lo + KPT)
                plsc.addupdate_scatter(cnt_v, [iv - lo], ones, mask=m)
                return carry

            lax.fori_loop(0, T // 16, hbody, 0)

            def ebody(j, acc):
                cv = cnt_v[pl.ds(j * 16, 16)]
                p = cv * (1.0 / T)
                t = p + 1e-10
                bits = plsc.bitcast(t, jnp.int32)
                ex = (bits >> 23) - 127
                mant = plsc.bitcast(
                    (bits & 0x007FFFFF) | 0x3F800000, jnp.float32)
                pol = jnp.full((16,), _LOG2_POLY[0], jnp.float32)
                for coef in _LOG2_POLY[1:]:
                    pol = pol * mant + jnp.float32(coef)
                lnt = jnp.float32(_LN2) * (ex.astype(jnp.float32) + pol)
                return acc + p * lnt

            acc = lax.fori_loop(0, KPT // 16, ebody,
                                jnp.zeros((16,), jnp.float32))
            stage_v[...] = acc
            pltpu.sync_copy(stage_v, shared.at[s])
            plsc.subcore_barrier()

            @pl.when(s == 0)
            def _():
                pltpu.sync_copy(shared, part_v)
                tot = jnp.zeros((16,), jnp.float32)
                for r in range(16):
                    tot = tot + part_v[r, :]
                ssum = jnp.sum(tot)
                pv = jnp.exp(jnp.zeros((16,), jnp.float32) - ssum)
                stage_v[...] = pv
                pltpu.sync_copy(stage_v, ppl_hbm)

    return _sc_body


def kernel(z, embedding):
    (idx,) = _vq_call(z, embedding)
    z_q, loss, ppl = _sc_call()(embedding, idx, z)
    return (z_q, loss[0], idx, ppl[0])


# speedup vs baseline: 1.1490x; 1.1490x over previous
import functools

import jax
import jax.numpy as jnp
from jax import lax
from jax.experimental import pallas as pl
from jax.experimental.pallas import tpu as pltpu
from jax.experimental.pallas import tpu_sc as plsc

K = 8192
E = 32
T = 4096
BT = 512
NT = T // BT
CC = 0.25


def _vq_body(z_ref, et_ref, idx_ref):
    i = pl.program_id(0)
    z = z_ref[...]
    et = et_ref[...]
    mm = lax.dot_general(z * -2.0, et, (((1,), (0,)), ((), ())),
                         preferred_element_type=jnp.float32)
    z2 = jnp.sum(z * z, axis=1, keepdims=True)
    e2 = jnp.sum(et * et, axis=0)
    scores = z2 + e2[None, :] + mm
    idx = jnp.argmin(scores, axis=1).astype(jnp.int32)
    idx_ref[pl.ds(i * BT, BT)] = idx


_vq_call = pl.pallas_call(
    _vq_body,
    grid=(NT,),
    in_specs=[
        pl.BlockSpec((BT, E), lambda i: (i, 0)),
        pl.BlockSpec((E, K), lambda i: (0, 0)),
    ],
    out_specs=[pl.BlockSpec((T,), lambda i: (0,))],
    out_shape=[jax.ShapeDtypeStruct((T,), jnp.int32)],
)


_NC, _NS = 2, 16
_TPS = T // _NS
KPT = K // _NS

_LOG2_POLY = (0.04342868488885992, -0.4048646480119264, 1.5938912717255367,
              -3.4924755918040757, 5.04685963461102, -2.7868074212476523)
_LN2 = 0.6931471805599453


@functools.cache
def _sc_call():
    mesh = plsc.VectorSubcoreMesh(
        core_axis_name="c", subcore_axis_name="s", num_cores=_NC)

    @functools.partial(
        pl.kernel,
        mesh=mesh,
        compiler_params=pltpu.CompilerParams(
            use_tc_tiling_on_sc=False, needs_layout_passes=False),
        out_type=(
            jax.ShapeDtypeStruct((T, E), jnp.float32),
            jax.ShapeDtypeStruct((16,), jnp.float32),
            jax.ShapeDtypeStruct((16,), jnp.float32),
        ),
        scratch_types=[
            pltpu.VMEM((_TPS,), jnp.int32),
            pltpu.VMEM((_TPS, E), jnp.float32),
            pltpu.VMEM((_TPS, E), jnp.float32),
            pltpu.VMEM((T,), jnp.int32),
            pltpu.VMEM((KPT,), jnp.float32),
            pltpu.VMEM((16,), jnp.float32),
            pltpu.VMEM((16, 16), jnp.float32),
            pltpu.VMEM_SHARED((16, 16), jnp.float32),
            pltpu.SemaphoreType.DMA,
        ],
    )
    def _sc_body(emb_hbm, idx_hbm, z_hbm, zq_hbm, loss_hbm, ppl_hbm,
                 idx_v, rows_v, zin_v, idxall_v, cnt_v, stage_v, part_v,
                 shared, sem):
        c = lax.axis_index("c")
        s = lax.axis_index("s")

        @pl.when(c == 1)
        def _():
            base = s * _TPS
            pltpu.sync_copy(idx_hbm.at[pl.ds(base, _TPS)], idx_v)
            pltpu.async_copy(emb_hbm.at[idx_v], rows_v, sem).wait()
            pltpu.sync_copy(rows_v, zq_hbm.at[pl.ds(base, _TPS)])
            pltpu.sync_copy(z_hbm.at[pl.ds(base, _TPS)], zin_v)

            def lbody(j, acc):
                d0 = rows_v[j, pl.ds(0, 16)] - zin_v[j, pl.ds(0, 16)]
                d1 = rows_v[j, pl.ds(16, 16)] - zin_v[j, pl.ds(16, 16)]
                return acc + d0 * d0 + d1 * d1

            acc = lax.fori_loop(0, _TPS, lbody, jnp.zeros((16,), jnp.float32))
            stage_v[...] = acc
            pltpu.sync_copy(stage_v, shared.at[s])
            plsc.subcore_barrier()

            @pl.when(s == 0)
            def _():
                pltpu.sync_copy(shared, part_v)
                tot = jnp.zeros((16,), jnp.float32)
                for r in range(16):
                    tot = tot + part_v[r, :]
                ssum = jnp.sum(tot)
                lv = jnp.zeros((16,), jnp.float32) + ssum * (CC / (T * E))
                stage_v[...] = lv
                pltpu.sync_copy(stage_v, loss_hbm)

        @pl.when(c == 0)
        def _():
            pltpu.sync_copy(idx_hbm, idxall_v)
            lo = s * KPT
            zeros = jnp.zeros((16,), jnp.float32)
            ones = jnp.full((16,), 1.0, jnp.float32)

            def zbody(j, carry):
                cnt_v[pl.ds(j * 16, 16)] = zeros
                return carry

            lax.fori_loop(0, KPT // 16, zbody, 0)

            def hbody(j, carry):
                iv = idxall_v[pl.ds(j * 16, 16)]
                m = (iv >= lo) & (iv < lo + KPT)
                plsc.addupdate_scatter(cnt_v, [iv - lo], ones, mask=m)
                return carry

            lax.fori_loop(0, T // 16, hbody, 0)

            def ebody(j, acc):
                cv = cnt_v[pl.ds(j * 16, 16)]
                p = cv * (1.0 / T)
                t = p + 1e-10
                bits = plsc.bitcast(t, jnp.int32)
                ex = (bits >> 23) - 127
                mant = plsc.bitcast(
                    (bits & 0x007FFFFF) | 0x3F800000, jnp.float32)
                pol = jnp.full((16,), _LOG2_POLY[0], jnp.float32)
                for coef in _LOG2_POLY[1:]:
                    pol = pol * mant + jnp.float32(coef)
                lnt = jnp.float32(_LN2) * (ex.astype(jnp.float32) + pol)
                return acc + p * lnt

            acc = lax.fori_loop(0, KPT // 16, ebody,
                                jnp.zeros((16,), jnp.float32))
            stage_v[...] = acc
            pltpu.sync_copy(stage_v, shared.at[s])
            plsc.subcore_barrier()

            @pl.when(s == 0)
            def _():
                pltpu.sync_copy(shared, part_v)
                tot = jnp.zeros((16,), jnp.float32)
                for r in range(16):
                    tot = tot + part_v[r, :]
                ssum = jnp.sum(tot)
                pv = jnp.exp(jnp.zeros((16,), jnp.float32) - ssum)
                stage_v[...] = pv
                pltpu.sync_copy(stage_v, ppl_hbm)

    return _sc_body


def kernel(z, embedding):
    (idx,) = _vq_call(z, embedding.T)
    z_q, loss, ppl = _sc_call()(embedding, idx, z)
    return (z_q, loss[0], idx, ppl[0])
